# baseline (speedup 1.0000x reference)
import jax
import jax.numpy as jnp
from jax import lax
from jax.experimental import pallas as pl
from jax.experimental.pallas import tpu as pltpu
from jax.experimental.pallas import tpu_sc as plsc

N = 10000
HID = 128
E = 320000
NC = 2
NS = 16
NW = NC * NS
LANES = 16

CHUNK = 128
CPT = 80
HALF = CPT // 2
ROWS_PAD = NW * CPT
EPAD = ROWS_PAD * CHUNK

ROWB = 80
TILE_ROWS = 640
NACC = N + ROWB

_mesh = plsc.VectorSubcoreMesh(
    core_axis_name="c", subcore_axis_name="s", num_cores=NC, num_subcores=NS
)


def _zero_fill(zbuf, width):
    def _fill(i, _):
        for j in range(width // LANES):
            zbuf[i, pl.ds(j * LANES, LANES)] = jnp.zeros((LANES,), jnp.float32)
        return 0

    lax.fori_loop(0, ROWB, _fill, 0)


def _zero_acc(zbuf, acc, s):
    row0 = s * TILE_ROWS
    nz = jnp.where(s == NS - 1, (NACC - (NS - 1) * TILE_ROWS) // ROWB,
                   TILE_ROWS // ROWB)

    def _zero(t, _):
        pltpu.sync_copy(zbuf, acc.at[pl.ds(row0 + t * ROWB, ROWB)])
        return 0

    lax.fori_loop(0, nz, _zero, 0)


def _dump_acc(acc, stage, out_hbm, c, s):
    row0 = s * TILE_ROWS
    nd = jnp.where(s == NS - 1, (N - (NS - 1) * TILE_ROWS) // ROWB,
                   TILE_ROWS // ROWB)

    def _dump(t, _):
        r0 = row0 + t * ROWB
        pltpu.sync_copy(acc.at[pl.ds(r0, ROWB)], stage)
        pltpu.sync_copy(stage, out_hbm.at[pl.ds(c * N + r0, ROWB)])
        return 0

    lax.fori_loop(0, nd, _dump, 0)


DCHUNK = 80
DCHUNKS_PER_TILE = E // (NW * DCHUNK)


def _degree_body(dst_hbm, cnt_hbm, dst_v, ones_v, zbuf, acc):
    c = lax.axis_index("c")
    s = lax.axis_index("s")
    w = c * NS + s

    _zero_fill(zbuf, HID)

    def _fill(i, _):
        for j in range(HID // LANES):
            ones_v[i, pl.ds(j * LANES, LANES)] = jnp.ones((LANES,), jnp.float32)
        return 0

    lax.fori_loop(0, DCHUNK, _fill, 0)

    _zero_acc(zbuf, acc, s)
    plsc.subcore_barrier()

    def _body(t, _):
        base = w * (DCHUNKS_PER_TILE * DCHUNK) + t * DCHUNK
        pltpu.sync_copy(dst_hbm.at[pl.ds(base, DCHUNK)], dst_v)
        pltpu.sync_copy(ones_v, acc.at[dst_v], add=True)
        return 0

    lax.fori_loop(0, DCHUNKS_PER_TILE, _body, 0)
    plsc.subcore_barrier()
    _dump_acc(acc, zbuf, cnt_hbm, c, s)


def _build_degree(interpret=False):
    return pl.kernel(
        _degree_body,
        out_type=jax.ShapeDtypeStruct((NC * N, HID), jnp.float32),
        mesh=_mesh,
        scratch_types=[
            pltpu.VMEM((DCHUNK,), jnp.int32),
            pltpu.VMEM((DCHUNK, HID), jnp.float32),
            pltpu.VMEM((ROWB, HID), jnp.float32),
            pltpu.VMEM_SHARED((NACC, HID), jnp.float32),
        ],
        interpret=interpret,
    )


def _edges_body(src_hbm, dst_hbm, hp_hbm, out_hbm,
                sidx, didx, rows_a, rows_b, acc, sem_a, sem_b):
    c = lax.axis_index("c")
    s = lax.axis_index("s")
    w = c * NS + s

    def _fillz(i, _):
        for j in range(HID // LANES):
            rows_a[i, pl.ds(j * LANES, LANES)] = jnp.zeros((LANES,), jnp.float32)
        return 0

    lax.fori_loop(0, ROWB, _fillz, 0)
    _zero_acc(rows_a.at[pl.ds(0, ROWB)], acc, s)
    plsc.subcore_barrier()

    for h in range(CPT // HALF):
        pltpu.sync_copy(src_hbm.at[pl.ds(w * CPT + h * HALF, HALF)], sidx)
        pltpu.sync_copy(dst_hbm.at[pl.ds(w * CPT + h * HALF, HALF)], didx)

        pltpu.async_copy(hp_hbm.at[sidx.at[0]], rows_a, sem_a)
        pltpu.async_copy(hp_hbm.at[sidx.at[1]], rows_b, sem_b)

        def _body(t, _):
            j0 = 2 * t
            j1 = j0 + 1
            pltpu.make_async_copy(hp_hbm.at[sidx.at[0]], rows_a, sem_a).wait()
            pltpu.sync_copy(rows_a, acc.at[didx.at[j0]], add=True)

            @pl.when(j0 + 2 < HALF)
            def _():
                pltpu.async_copy(hp_hbm.at[sidx.at[j0 + 2]], rows_a, sem_a)

            pltpu.make_async_copy(hp_hbm.at[sidx.at[1]], rows_b, sem_b).wait()
            pltpu.sync_copy(rows_b, acc.at[didx.at[j1]], add=True)

            @pl.when(j1 + 2 < HALF)
            def _():
                pltpu.async_copy(hp_hbm.at[sidx.at[j1 + 2]], rows_b, sem_b)

            return 0

        lax.fori_loop(0, HALF // 2, _body, 0)
    plsc.subcore_barrier()
    _dump_acc(acc, rows_a.at[pl.ds(0, ROWB)], out_hbm, c, s)


def _build_edges(interpret=False):
    return pl.kernel(
        _edges_body,
        out_type=jax.ShapeDtypeStruct((NC * N, HID), jnp.float32),
        mesh=_mesh,
        scratch_types=[
            pltpu.VMEM((HALF, CHUNK), jnp.int32),
            pltpu.VMEM((HALF, CHUNK), jnp.int32),
            pltpu.VMEM((CHUNK, HID), jnp.float32),
            pltpu.VMEM((CHUNK, HID), jnp.float32),
            pltpu.VMEM_SHARED((NACC, HID), jnp.float32),
            pltpu.SemaphoreType.DMA,
            pltpu.SemaphoreType.DMA,
        ],
        interpret=interpret,
    )


_degree_sc = _build_degree()
_edges_sc = _build_edges()


BR = 1000


def _dense1_body(x_ref, w_ref, cnt_ref, h_ref, dis_ref):
    deg = cnt_ref[0][:, 0] + cnt_ref[1][:, 0] + 1.0
    dis = lax.rsqrt(deg)[:, None]
    h = jnp.dot(x_ref[...], w_ref[...], preferred_element_type=jnp.float32)
    h_ref[...] = h * dis
    dis_ref[...] = dis


_dense1 = pl.pallas_call(
    _dense1_body,
    grid=(N // BR,),
    in_specs=[
        pl.BlockSpec((BR, HID), lambda i: (i, 0)),
        pl.BlockSpec((HID, HID), lambda i: (0, 0)),
        pl.BlockSpec((NC, BR, HID), lambda i: (0, i, 0)),
    ],
    out_specs=[
        pl.BlockSpec((BR, HID), lambda i: (i, 0)),
        pl.BlockSpec((BR, 1), lambda i: (i, 0)),
    ],
    out_shape=[
        jax.ShapeDtypeStruct((N, HID), jnp.float32),
        jax.ShapeDtypeStruct((N, 1), jnp.float32),
    ],
)


def _dense2_body(a_ref, hp_ref, dis_ref, b_ref, w_ref, h_ref):
    dis = dis_ref[...]
    seg = a_ref[0] + a_ref[1] + hp_ref[...]
    x2 = jnp.maximum(seg * dis + b_ref[...], 0.0)
    h_ref[...] = jnp.dot(x2, w_ref[...], preferred_element_type=jnp.float32) * dis


_dense2 = pl.pallas_call(
    _dense2_body,
    grid=(N // BR,),
    in_specs=[
        pl.BlockSpec((NC, BR, HID), lambda i: (0, i, 0)),
        pl.BlockSpec((BR, HID), lambda i: (i, 0)),
        pl.BlockSpec((BR, 1), lambda i: (i, 0)),
        pl.BlockSpec((1, HID), lambda i: (0, 0)),
        pl.BlockSpec((HID, HID), lambda i: (0, 0)),
    ],
    out_specs=pl.BlockSpec((BR, HID), lambda i: (i, 0)),
    out_shape=jax.ShapeDtypeStruct((N, HID), jnp.float32),
)


def _dense3_body(a_ref, hp_ref, dis_ref, b_ref, o_ref):
    seg = a_ref[0] + a_ref[1] + hp_ref[...]
    o_ref[...] = jnp.maximum(seg * dis_ref[...] + b_ref[...], 0.0)


_dense3 = pl.pallas_call(
    _dense3_body,
    grid=(N // BR,),
    in_specs=[
        pl.BlockSpec((NC, BR, HID), lambda i: (0, i, 0)),
        pl.BlockSpec((BR, HID), lambda i: (i, 0)),
        pl.BlockSpec((BR, 1), lambda i: (i, 0)),
        pl.BlockSpec((1, HID), lambda i: (0, 0)),
    ],
    out_specs=pl.BlockSpec((BR, HID), lambda i: (i, 0)),
    out_shape=jax.ShapeDtypeStruct((N, HID), jnp.float32),
)


def kernel(edge_index, node_features, W1, b1, W2, b2):
    ei = edge_index.astype(jnp.int32)
    pad = EPAD - E
    sink = N + jnp.arange(pad, dtype=jnp.int32) % ROWB
    src = jnp.concatenate([ei[0], jnp.zeros((pad,), jnp.int32)]).reshape(ROWS_PAD, CHUNK)
    dst = jnp.concatenate([ei[1], sink]).reshape(ROWS_PAD, CHUNK)
    cnt = _degree_sc(ei[1]).reshape(NC, N, HID)
    h1p, dis = _dense1(node_features, W1, cnt)
    a1 = _edges_sc(src, dst, h1p).reshape(NC, N, HID)
    h2p = _dense2(a1, h1p, dis, b1.reshape(1, HID), W2)
    a2 = _edges_sc(src, dst, h2p).reshape(NC, N, HID)
    return _dense3(a2, h2p, dis, b2.reshape(1, HID))


# speedup vs baseline: 2.7255x; 2.7255x over previous
import jax
import jax.numpy as jnp
from jax import lax
from jax.experimental import pallas as pl
from jax.experimental.pallas import tpu as pltpu
from jax.experimental.pallas import tpu_sc as plsc

N = 10000
HID = 128
E = 320000
NC = 2
NS = 16
NW = NC * NS
LANES = 16

CHUNK = 128
CPT = 80
HALF = CPT // 2
ROWS_PAD = NW * CPT
EPAD = ROWS_PAD * CHUNK

ROWB = 80
TILE_ROWS = 640
NACC = N + ROWB

_mesh = plsc.VectorSubcoreMesh(
    core_axis_name="c", subcore_axis_name="s", num_cores=NC, num_subcores=NS
)


def _zero_fill(zbuf, width):
    def _fill(i, _):
        for j in range(width // LANES):
            zbuf[i, pl.ds(j * LANES, LANES)] = jnp.zeros((LANES,), jnp.float32)
        return 0

    lax.fori_loop(0, ROWB, _fill, 0)


def _zero_acc(zbuf, acc, s):
    row0 = s * TILE_ROWS
    nz = jnp.where(s == NS - 1, (NACC - (NS - 1) * TILE_ROWS) // ROWB,
                   TILE_ROWS // ROWB)

    def _zero(t, _):
        pltpu.sync_copy(zbuf, acc.at[pl.ds(row0 + t * ROWB, ROWB)])
        return 0

    lax.fori_loop(0, nz, _zero, 0)


def _dump_acc(acc, stage, out_hbm, c, s):
    row0 = s * TILE_ROWS
    nd = jnp.where(s == NS - 1, (N - (NS - 1) * TILE_ROWS) // ROWB,
                   TILE_ROWS // ROWB)

    def _dump(t, _):
        r0 = row0 + t * ROWB
        pltpu.sync_copy(acc.at[pl.ds(r0, ROWB)], stage)
        pltpu.sync_copy(stage, out_hbm.at[pl.ds(c * N + r0, ROWB)])
        return 0

    lax.fori_loop(0, nd, _dump, 0)


DCHUNK = 80
DCHUNKS_PER_TILE = E // (NW * DCHUNK)


def _degree_body(dst_hbm, cnt_hbm, dst_v, ones_v, zbuf, acc):
    c = lax.axis_index("c")
    s = lax.axis_index("s")
    w = c * NS + s

    _zero_fill(zbuf, HID)

    def _fill(i, _):
        for j in range(HID // LANES):
            ones_v[i, pl.ds(j * LANES, LANES)] = jnp.ones((LANES,), jnp.float32)
        return 0

    lax.fori_loop(0, DCHUNK, _fill, 0)

    _zero_acc(zbuf, acc, s)
    plsc.subcore_barrier()

    def _body(t, _):
        base = w * (DCHUNKS_PER_TILE * DCHUNK) + t * DCHUNK
        pltpu.sync_copy(dst_hbm.at[pl.ds(base, DCHUNK)], dst_v)
        pltpu.sync_copy(ones_v, acc.at[dst_v], add=True)
        return 0

    lax.fori_loop(0, DCHUNKS_PER_TILE, _body, 0)
    plsc.subcore_barrier()
    _dump_acc(acc, zbuf, cnt_hbm, c, s)


def _build_degree(interpret=False):
    return pl.kernel(
        _degree_body,
        out_type=jax.ShapeDtypeStruct((NC * N, HID), jnp.float32),
        mesh=_mesh,
        scratch_types=[
            pltpu.VMEM((DCHUNK,), jnp.int32),
            pltpu.VMEM((DCHUNK, HID), jnp.float32),
            pltpu.VMEM((ROWB, HID), jnp.float32),
            pltpu.VMEM_SHARED((NACC, HID), jnp.float32),
        ],
        interpret=interpret,
    )


def _edges_body(src_hbm, dst_hbm, hp_hbm, out_hbm,
                sidx, didx, rows_a, rows_b, acc, sem_a, sem_b):
    c = lax.axis_index("c")
    s = lax.axis_index("s")
    w = c * NS + s

    def _fillz(i, _):
        for j in range(HID // LANES):
            rows_a[i, pl.ds(j * LANES, LANES)] = jnp.zeros((LANES,), jnp.float32)
        return 0

    lax.fori_loop(0, ROWB, _fillz, 0)
    _zero_acc(rows_a.at[pl.ds(0, ROWB)], acc, s)
    plsc.subcore_barrier()

    for h in range(CPT // HALF):
        pltpu.sync_copy(src_hbm.at[pl.ds(w * CPT + h * HALF, HALF)], sidx)
        pltpu.sync_copy(dst_hbm.at[pl.ds(w * CPT + h * HALF, HALF)], didx)

        pltpu.async_copy(hp_hbm.at[sidx.at[0]], rows_a, sem_a)
        pltpu.async_copy(hp_hbm.at[sidx.at[1]], rows_b, sem_b)

        def _body(t, _):
            j0 = 2 * t
            j1 = j0 + 1
            pltpu.make_async_copy(hp_hbm.at[sidx.at[0]], rows_a, sem_a).wait()
            pltpu.sync_copy(rows_a, acc.at[didx.at[j0]], add=True)

            @pl.when(j0 + 2 < HALF)
            def _():
                pltpu.async_copy(hp_hbm.at[sidx.at[j0 + 2]], rows_a, sem_a)

            pltpu.make_async_copy(hp_hbm.at[sidx.at[1]], rows_b, sem_b).wait()
            pltpu.sync_copy(rows_b, acc.at[didx.at[j1]], add=True)

            @pl.when(j1 + 2 < HALF)
            def _():
                pltpu.async_copy(hp_hbm.at[sidx.at[j1 + 2]], rows_b, sem_b)

            return 0

        lax.fori_loop(0, HALF // 2, _body, 0)
    plsc.subcore_barrier()
    _dump_acc(acc, rows_a.at[pl.ds(0, ROWB)], out_hbm, c, s)


def _build_edges(interpret=False):
    return pl.kernel(
        _edges_body,
        out_type=jax.ShapeDtypeStruct((NC * N, HID), jnp.float32),
        mesh=_mesh,
        scratch_types=[
            pltpu.VMEM((HALF, CHUNK), jnp.int32),
            pltpu.VMEM((HALF, CHUNK), jnp.int32),
            pltpu.VMEM((CHUNK, HID), jnp.float32),
            pltpu.VMEM((CHUNK, HID), jnp.float32),
            pltpu.VMEM_SHARED((NACC, HID), jnp.float32),
            pltpu.SemaphoreType.DMA,
            pltpu.SemaphoreType.DMA,
        ],
        interpret=interpret,
    )


_degree_sc = _build_degree()
_edges_sc = _build_edges()


BR = 1000


def _dense1_body(x_ref, w_ref, cnt_ref, h_ref, dis_ref):
    deg = cnt_ref[0][:, 0] + cnt_ref[1][:, 0] + 1.0
    dis = lax.rsqrt(deg)[:, None]
    h = jnp.dot(x_ref[...], w_ref[...], preferred_element_type=jnp.float32)
    h_ref[...] = h * dis
    dis_ref[...] = dis


_dense1 = pl.pallas_call(
    _dense1_body,
    grid=(N // BR,),
    in_specs=[
        pl.BlockSpec((BR, HID), lambda i: (i, 0)),
        pl.BlockSpec((HID, HID), lambda i: (0, 0)),
        pl.BlockSpec((NC, BR, HID), lambda i: (0, i, 0)),
    ],
    out_specs=[
        pl.BlockSpec((BR, HID), lambda i: (i, 0)),
        pl.BlockSpec((BR, 1), lambda i: (i, 0)),
    ],
    out_shape=[
        jax.ShapeDtypeStruct((N, HID), jnp.float32),
        jax.ShapeDtypeStruct((N, 1), jnp.float32),
    ],
)


def _dense2_body(a_ref, hp_ref, dis_ref, b_ref, w_ref, h_ref):
    dis = dis_ref[...]
    seg = a_ref[0] + a_ref[1] + hp_ref[...]
    x2 = jnp.maximum(seg * dis + b_ref[...], 0.0)
    h_ref[...] = jnp.dot(x2, w_ref[...], preferred_element_type=jnp.float32) * dis


_dense2 = pl.pallas_call(
    _dense2_body,
    grid=(N // BR,),
    in_specs=[
        pl.BlockSpec((NC, BR, HID), lambda i: (0, i, 0)),
        pl.BlockSpec((BR, HID), lambda i: (i, 0)),
        pl.BlockSpec((BR, 1), lambda i: (i, 0)),
        pl.BlockSpec((1, HID), lambda i: (0, 0)),
        pl.BlockSpec((HID, HID), lambda i: (0, 0)),
    ],
    out_specs=pl.BlockSpec((BR, HID), lambda i: (i, 0)),
    out_shape=jax.ShapeDtypeStruct((N, HID), jnp.float32),
)


def _dense3_body(a_ref, hp_ref, dis_ref, b_ref, o_ref):
    seg = a_ref[0] + a_ref[1] + hp_ref[...]
    o_ref[...] = jnp.maximum(seg * dis_ref[...] + b_ref[...], 0.0)


_dense3 = pl.pallas_call(
    _dense3_body,
    grid=(N // BR,),
    in_specs=[
        pl.BlockSpec((NC, BR, HID), lambda i: (0, i, 0)),
        pl.BlockSpec((BR, HID), lambda i: (i, 0)),
        pl.BlockSpec((BR, 1), lambda i: (i, 0)),
        pl.BlockSpec((1, HID), lambda i: (0, 0)),
    ],
    out_specs=pl.BlockSpec((BR, HID), lambda i: (i, 0)),
    out_shape=jax.ShapeDtypeStruct((N, HID), jnp.float32),
)


def kernel(edge_index, node_features, W1, b1, W2, b2):
    ei = edge_index.astype(jnp.int32)
    pad = EPAD - E
    fill = jnp.arange(pad, dtype=jnp.int32)
    src = jnp.concatenate([ei[0], fill % N]).reshape(ROWS_PAD, CHUNK)
    dst = jnp.concatenate([ei[1], N + fill % ROWB]).reshape(ROWS_PAD, CHUNK)
    cnt = _degree_sc(ei[1]).reshape(NC, N, HID)
    h1p, dis = _dense1(node_features, W1, cnt)
    a1 = _edges_sc(src, dst, h1p).reshape(NC, N, HID)
    h2p = _dense2(a1, h1p, dis, b1.reshape(1, HID), W2)
    a2 = _edges_sc(src, dst, h2p).reshape(NC, N, HID)
    return _dense3(a2, h2p, dis, b2.reshape(1, HID))


# speedup vs baseline: 3.0896x; 1.1336x over previous
import jax
import jax.numpy as jnp
import numpy as np
from jax import lax
from jax.experimental import pallas as pl
from jax.experimental.pallas import tpu as pltpu
from jax.experimental.pallas import tpu_sc as plsc

N = 10000
HID = 128
E = 320000
NC = 2
NS = 16
NW = NC * NS
LANES = 16

CHUNK = 128
CPT = 80
HALF = CPT // 2
ROWS_PAD = NW * CPT
EPAD = ROWS_PAD * CHUNK

ROWB = 80
TILE_ROWS = 640
NACC = N + ROWB

_mesh = plsc.VectorSubcoreMesh(
    core_axis_name="c", subcore_axis_name="s", num_cores=NC, num_subcores=NS
)


def _zero_fill(zbuf, width):
    def _fill(i, _):
        for j in range(width // LANES):
            zbuf[i, pl.ds(j * LANES, LANES)] = jnp.zeros((LANES,), jnp.float32)
        return 0

    lax.fori_loop(0, ROWB, _fill, 0)


def _zero_acc(zbuf, acc, s):
    row0 = s * TILE_ROWS
    nz = jnp.where(s == NS - 1, (NACC - (NS - 1) * TILE_ROWS) // ROWB,
                   TILE_ROWS // ROWB)

    def _zero(t, _):
        pltpu.sync_copy(zbuf, acc.at[pl.ds(row0 + t * ROWB, ROWB)])
        return 0

    lax.fori_loop(0, nz, _zero, 0)


def _dump_acc(acc, stage, out_hbm, c, s):
    row0 = s * TILE_ROWS
    nd = jnp.where(s == NS - 1, (N - (NS - 1) * TILE_ROWS) // ROWB,
                   TILE_ROWS // ROWB)

    def _dump(t, _):
        r0 = row0 + t * ROWB
        pltpu.sync_copy(acc.at[pl.ds(r0, ROWB)], stage)
        pltpu.sync_copy(stage, out_hbm.at[pl.ds(c * N + r0, ROWB)])
        return 0

    lax.fori_loop(0, nd, _dump, 0)


def _degree_body(dst_hbm, cnt_hbm, didx, ones_v, zbuf, acc):
    c = lax.axis_index("c")
    s = lax.axis_index("s")
    w = c * NS + s

    _zero_fill(zbuf, HID)

    def _fill(i, _):
        for j in range(HID // LANES):
            ones_v[i, pl.ds(j * LANES, LANES)] = jnp.ones((LANES,), jnp.float32)
        return 0

    lax.fori_loop(0, CHUNK, _fill, 0)

    _zero_acc(zbuf, acc, s)
    plsc.subcore_barrier()

    for h in range(CPT // HALF):
        pltpu.sync_copy(dst_hbm.at[pl.ds(w * CPT + h * HALF, HALF)], didx)

        def _body(t, _):
            pltpu.sync_copy(ones_v, acc.at[didx.at[t]], add=True)
            return 0

        lax.fori_loop(0, HALF, _body, 0)
    plsc.subcore_barrier()
    _dump_acc(acc, zbuf, cnt_hbm, c, s)


def _build_degree(interpret=False):
    return pl.kernel(
        _degree_body,
        out_type=jax.ShapeDtypeStruct((NC * N, HID), jnp.float32),
        mesh=_mesh,
        scratch_types=[
            pltpu.VMEM((HALF, CHUNK), jnp.int32),
            pltpu.VMEM((CHUNK, HID), jnp.float32),
            pltpu.VMEM((ROWB, HID), jnp.float32),
            pltpu.VMEM_SHARED((NACC, HID), jnp.float32),
        ],
        interpret=interpret,
    )


def _edges_body(src_hbm, dst_hbm, hp_hbm, out_hbm,
                sidx, didx, rows_a, rows_b, acc, sem_a, sem_b):
    c = lax.axis_index("c")
    s = lax.axis_index("s")
    w = c * NS + s

    def _fillz(i, _):
        for j in range(HID // LANES):
            rows_a[i, pl.ds(j * LANES, LANES)] = jnp.zeros((LANES,), jnp.float32)
        return 0

    lax.fori_loop(0, ROWB, _fillz, 0)
    _zero_acc(rows_a.at[pl.ds(0, ROWB)], acc, s)
    plsc.subcore_barrier()

    for h in range(CPT // HALF):
        pltpu.sync_copy(src_hbm.at[pl.ds(w * CPT + h * HALF, HALF)], sidx)
        pltpu.sync_copy(dst_hbm.at[pl.ds(w * CPT + h * HALF, HALF)], didx)

        pltpu.async_copy(hp_hbm.at[sidx.at[0]], rows_a, sem_a)
        pltpu.async_copy(hp_hbm.at[sidx.at[1]], rows_b, sem_b)

        def _body(t, _):
            j0 = 2 * t
            j1 = j0 + 1
            pltpu.make_async_copy(hp_hbm.at[sidx.at[0]], rows_a, sem_a).wait()
            pltpu.sync_copy(rows_a, acc.at[didx.at[j0]], add=True)

            @pl.when(j0 + 2 < HALF)
            def _():
                pltpu.async_copy(hp_hbm.at[sidx.at[j0 + 2]], rows_a, sem_a)

            pltpu.make_async_copy(hp_hbm.at[sidx.at[1]], rows_b, sem_b).wait()
            pltpu.sync_copy(rows_b, acc.at[didx.at[j1]], add=True)

            @pl.when(j1 + 2 < HALF)
            def _():
                pltpu.async_copy(hp_hbm.at[sidx.at[j1 + 2]], rows_b, sem_b)

            return 0

        lax.fori_loop(0, HALF // 2, _body, 0)
    plsc.subcore_barrier()
    _dump_acc(acc, rows_a.at[pl.ds(0, ROWB)], out_hbm, c, s)


def _build_edges(interpret=False):
    return pl.kernel(
        _edges_body,
        out_type=jax.ShapeDtypeStruct((NC * N, HID), jnp.float32),
        mesh=_mesh,
        scratch_types=[
            pltpu.VMEM((HALF, CHUNK), jnp.int32),
            pltpu.VMEM((HALF, CHUNK), jnp.int32),
            pltpu.VMEM((CHUNK, HID), jnp.float32),
            pltpu.VMEM((CHUNK, HID), jnp.float32),
            pltpu.VMEM_SHARED((NACC, HID), jnp.float32),
            pltpu.SemaphoreType.DMA,
            pltpu.SemaphoreType.DMA,
        ],
        interpret=interpret,
    )


_degree_sc = _build_degree()
_edges_sc = _build_edges()


BR = 1000


def _mm1_body(x_ref, w_ref, g_ref):
    g_ref[...] = jnp.dot(x_ref[...], w_ref[...],
                         preferred_element_type=jnp.float32)


_mm1 = pl.pallas_call(
    _mm1_body,
    grid=(N // BR,),
    in_specs=[
        pl.BlockSpec((BR, HID), lambda i: (i, 0)),
        pl.BlockSpec((HID, HID), lambda i: (0, 0)),
    ],
    out_specs=pl.BlockSpec((BR, HID), lambda i: (i, 0)),
    out_shape=jax.ShapeDtypeStruct((N, HID), jnp.float32),
)


def _scale1_body(g_ref, cnt_ref, h_ref, dis_ref):
    deg = cnt_ref[0][:, 0] + cnt_ref[1][:, 0] + 1.0
    dis = lax.rsqrt(deg)[:, None]
    h_ref[...] = g_ref[...] * dis
    dis_ref[...] = dis


_scale1 = pl.pallas_call(
    _scale1_body,
    grid=(N // BR,),
    in_specs=[
        pl.BlockSpec((BR, HID), lambda i: (i, 0)),
        pl.BlockSpec((NC, BR, HID), lambda i: (0, i, 0)),
    ],
    out_specs=[
        pl.BlockSpec((BR, HID), lambda i: (i, 0)),
        pl.BlockSpec((BR, 1), lambda i: (i, 0)),
    ],
    out_shape=[
        jax.ShapeDtypeStruct((N, HID), jnp.float32),
        jax.ShapeDtypeStruct((N, 1), jnp.float32),
    ],
)


def _dense2_body(a_ref, hp_ref, dis_ref, b_ref, w_ref, h_ref):
    dis = dis_ref[...]
    seg = a_ref[0] + a_ref[1] + hp_ref[...]
    x2 = jnp.maximum(seg * dis + b_ref[...], 0.0)
    h_ref[...] = jnp.dot(x2, w_ref[...], preferred_element_type=jnp.float32) * dis


_dense2 = pl.pallas_call(
    _dense2_body,
    grid=(N // BR,),
    in_specs=[
        pl.BlockSpec((NC, BR, HID), lambda i: (0, i, 0)),
        pl.BlockSpec((BR, HID), lambda i: (i, 0)),
        pl.BlockSpec((BR, 1), lambda i: (i, 0)),
        pl.BlockSpec((1, HID), lambda i: (0, 0)),
        pl.BlockSpec((HID, HID), lambda i: (0, 0)),
    ],
    out_specs=pl.BlockSpec((BR, HID), lambda i: (i, 0)),
    out_shape=jax.ShapeDtypeStruct((N, HID), jnp.float32),
)


def _dense3_body(a_ref, hp_ref, dis_ref, b_ref, o_ref):
    seg = a_ref[0] + a_ref[1] + hp_ref[...]
    o_ref[...] = jnp.maximum(seg * dis_ref[...] + b_ref[...], 0.0)


_dense3 = pl.pallas_call(
    _dense3_body,
    grid=(N // BR,),
    in_specs=[
        pl.BlockSpec((NC, BR, HID), lambda i: (0, i, 0)),
        pl.BlockSpec((BR, HID), lambda i: (i, 0)),
        pl.BlockSpec((BR, 1), lambda i: (i, 0)),
        pl.BlockSpec((1, HID), lambda i: (0, 0)),
    ],
    out_specs=pl.BlockSpec((BR, HID), lambda i: (i, 0)),
    out_shape=jax.ShapeDtypeStruct((N, HID), jnp.float32),
)


_PAD = EPAD - E
_FILL_SRC = np.arange(_PAD, dtype=np.int32) % N
_FILL_DST = N + np.arange(_PAD, dtype=np.int32) % ROWB


def kernel(edge_index, node_features, W1, b1, W2, b2):
    ei = edge_index.astype(jnp.int32)
    src = jnp.concatenate([ei[0], jnp.asarray(_FILL_SRC)]).reshape(ROWS_PAD, CHUNK)
    dst = jnp.concatenate([ei[1], jnp.asarray(_FILL_DST)]).reshape(ROWS_PAD, CHUNK)
    cnt = _degree_sc(dst).reshape(NC, N, HID)
    g1 = _mm1(node_features, W1)
    h1p, dis = _scale1(g1, cnt)
    a1 = _edges_sc(src, dst, h1p).reshape(NC, N, HID)
    h2p = _dense2(a1, h1p, dis, b1.reshape(1, HID), W2)
    a2 = _edges_sc(src, dst, h2p).reshape(NC, N, HID)
    return _dense3(a2, h2p, dis, b2.reshape(1, HID))
